# Initial kernel scaffold; baseline (speedup 1.0000x reference)
import jax
import jax.numpy as jnp
from jax.experimental import pallas as pl


def kernel(x, edge_index, batch, W1, a_src1, a_dst1, b1, W2, a_src2, a_dst2, b2, W_res, b_res, W_fc, b_fc):
    raise NotImplementedError("write your pallas kernel here")


# speedup vs baseline: 9.9857x; 9.9857x over previous
import functools
import jax
import jax.numpy as jnp
from jax import lax
from jax.experimental import pallas as pl
from jax.experimental.pallas import tpu as pltpu
from jax.experimental.pallas import tpu_sc as plsc

N = 10000
E = 320000
D = 128
H = 4
C = 64
HC = 256
G = 64
OUT = 128

NP = 10240
BN = 1024
NB = NP // BN


def _stage_a_body(x_ref, wcat_ref, bcat_ref, amat_ref, oh_ref, aa_ref, exl_ref):
    xb = x_ref[...]
    hres = jnp.dot(xb, wcat_ref[...], preferred_element_type=jnp.float32)
    hres = hres + bcat_ref[...]
    oh_ref[...] = hres
    aa = jnp.dot(hres[:, :HC], amat_ref[...], preferred_element_type=jnp.float32)
    aa_ref[...] = aa
    s = aa[:, :H] + aa[:, H:]
    s = jnp.where(s >= 0, s, 0.2 * s)
    exl_ref[...] = jnp.exp(s)


def _stage_a(x_p, Wcat, bcat, amat):
    return pl.pallas_call(
        _stage_a_body,
        grid=(NB,),
        in_specs=[
            pl.BlockSpec((BN, D), lambda i: (i, 0)),
            pl.BlockSpec((D, 2 * HC), lambda i: (0, 0)),
            pl.BlockSpec((1, 2 * HC), lambda i: (0, 0)),
            pl.BlockSpec((HC, 2 * H), lambda i: (0, 0)),
        ],
        out_specs=[
            pl.BlockSpec((BN, 2 * HC), lambda i: (i, 0)),
            pl.BlockSpec((BN, 2 * H), lambda i: (i, 0)),
            pl.BlockSpec((BN, H), lambda i: (i, 0)),
        ],
        out_shape=[
            jax.ShapeDtypeStruct((NP, 2 * HC), jnp.float32),
            jax.ShapeDtypeStruct((NP, 2 * H), jnp.float32),
            jax.ShapeDtypeStruct((NP, H), jnp.float32),
        ],
    )(x_p, Wcat, bcat, amat)


def _stage_b_body(agg_ref, h1_ref, exl_ref, den_ref, b1_ref, rmat_ref,
                  w2_ref, amat_ref, oh_ref, aa_ref, exl2_ref):
    exl_rep = jnp.dot(exl_ref[...], rmat_ref[...], preferred_element_type=jnp.float32)
    den_rep = jnp.dot(den_ref[...], rmat_ref[...], preferred_element_type=jnp.float32)
    num = agg_ref[...] + exl_rep * h1_ref[...]
    o1 = jnp.maximum(num / den_rep + b1_ref[...], 0.0)
    h2 = jnp.dot(o1, w2_ref[...], preferred_element_type=jnp.float32)
    oh_ref[...] = h2
    aa = jnp.dot(h2, amat_ref[...], preferred_element_type=jnp.float32)
    aa_ref[...] = aa
    s = aa[:, :H] + aa[:, H:]
    s = jnp.where(s >= 0, s, 0.2 * s)
    exl2_ref[...] = jnp.exp(s)


def _stage_b(agg, h1, exl, den, b1r, rmat, W2, amat):
    return pl.pallas_call(
        _stage_b_body,
        grid=(NB,),
        in_specs=[
            pl.BlockSpec((BN, HC), lambda i: (i, 0)),
            pl.BlockSpec((BN, HC), lambda i: (i, 0)),
            pl.BlockSpec((BN, H), lambda i: (i, 0)),
            pl.BlockSpec((BN, H), lambda i: (i, 0)),
            pl.BlockSpec((1, HC), lambda i: (0, 0)),
            pl.BlockSpec((H, HC), lambda i: (0, 0)),
            pl.BlockSpec((HC, HC), lambda i: (0, 0)),
            pl.BlockSpec((HC, 2 * H), lambda i: (0, 0)),
        ],
        out_specs=[
            pl.BlockSpec((BN, HC), lambda i: (i, 0)),
            pl.BlockSpec((BN, 2 * H), lambda i: (i, 0)),
            pl.BlockSpec((BN, H), lambda i: (i, 0)),
        ],
        out_shape=[
            jax.ShapeDtypeStruct((NP, HC), jnp.float32),
            jax.ShapeDtypeStruct((NP, 2 * H), jnp.float32),
            jax.ShapeDtypeStruct((NP, H), jnp.float32),
        ],
    )(agg, h1, exl, den, b1r, rmat, W2, amat)


def _stage_c_body(agg_ref, h2_ref, exl_ref, den_ref, res_ref, ohw_ref, b2_ref,
                  rmat_ref, wfc_ref, bfc_ref, out_ref, acc_ref):
    i = pl.program_id(0)

    @pl.when(i == 0)
    def _():
        acc_ref[...] = jnp.zeros_like(acc_ref)

    exl_rep = jnp.dot(exl_ref[...], rmat_ref[...], preferred_element_type=jnp.float32)
    den_rep = jnp.dot(den_ref[...], rmat_ref[...], preferred_element_type=jnp.float32)
    num = agg_ref[...] + exl_rep * h2_ref[...]
    hfin = jnp.maximum(num / den_rep + b2_ref[...], 0.0) + res_ref[...]
    acc_ref[...] += jnp.dot(ohw_ref[...], hfin, preferred_element_type=jnp.float32)

    @pl.when(i == NB - 1)
    def _():
        logits = jnp.dot(acc_ref[...], wfc_ref[...],
                         preferred_element_type=jnp.float32) + bfc_ref[...]
        m = jnp.max(logits, axis=1, keepdims=True)
        lse = jnp.log(jnp.sum(jnp.exp(logits - m), axis=1, keepdims=True)) + m
        out_ref[...] = logits - lse


def _stage_c(agg, h2, exl, den, res, ohw, b2r, rmat, Wfc, bfcr):
    return pl.pallas_call(
        _stage_c_body,
        grid=(NB,),
        in_specs=[
            pl.BlockSpec((BN, HC), lambda i: (i, 0)),
            pl.BlockSpec((BN, HC), lambda i: (i, 0)),
            pl.BlockSpec((BN, H), lambda i: (i, 0)),
            pl.BlockSpec((BN, H), lambda i: (i, 0)),
            pl.BlockSpec((BN, HC), lambda i: (i, 0)),
            pl.BlockSpec((G, BN), lambda i: (0, i)),
            pl.BlockSpec((1, HC), lambda i: (0, 0)),
            pl.BlockSpec((H, HC), lambda i: (0, 0)),
            pl.BlockSpec((HC, OUT), lambda i: (0, 0)),
            pl.BlockSpec((1, OUT), lambda i: (0, 0)),
        ],
        out_specs=pl.BlockSpec((G, OUT), lambda i: (0, 0)),
        out_shape=jax.ShapeDtypeStruct((G, OUT), jnp.float32),
        scratch_shapes=[pltpu.VMEM((G, HC), jnp.float32)],
    )(agg, h2, exl, den, res, ohw, b2r, rmat, Wfc, bfcr)


def _gat_edge_sc(als, ald, exl, h_heads, src, dst):
    e = als[src] + ald[dst]
    e = jnp.where(e >= 0, e, 0.2 * e)
    ex = jnp.exp(e)
    den = jax.ops.segment_sum(ex, dst, num_segments=NP) + exl
    msg = h_heads[src] * jnp.repeat(ex, C, axis=1)
    agg = jax.ops.segment_sum(msg, dst, num_segments=NP)
    return agg, den


def kernel(x, edge_index, batch, W1, a_src1, a_dst1, b1, W2, a_src2, a_dst2,
           b2, W_res, b_res, W_fc, b_fc):
    f32 = jnp.float32
    x_p = jnp.zeros((NP, D), f32).at[:N].set(x)
    src = edge_index[0]
    dst = edge_index[1]

    eyeC = jnp.eye(H, dtype=f32)
    amat1 = jnp.concatenate(
        [jnp.einsum('hc,hk->hck', a_src1, eyeC).reshape(HC, H),
         jnp.einsum('hc,hk->hck', a_dst1, eyeC).reshape(HC, H)], axis=1)
    amat2 = jnp.concatenate(
        [jnp.einsum('hc,hk->hck', a_src2, eyeC).reshape(HC, H),
         jnp.einsum('hc,hk->hck', a_dst2, eyeC).reshape(HC, H)], axis=1)
    rmat = jnp.repeat(jnp.eye(H, dtype=f32), C, axis=1)

    Wcat = jnp.concatenate([W1, W_res], axis=1)
    bcat = jnp.concatenate([jnp.zeros((HC,), f32), b_res])[None, :]

    oh, aa1, exl1 = _stage_a(x_p, Wcat, bcat, amat1)
    h1 = oh[:, :HC]
    res = oh[:, HC:]

    agg1_f, den1_f = _gat_edge_sc(aa1[:, :H], aa1[:, H:], exl1, h1, src, dst)

    h2, aa2, exl2 = _stage_b(agg1_f, h1, exl1, den1_f, b1[None, :], rmat,
                             W2, amat2)

    agg2_f, den2_f = _gat_edge_sc(aa2[:, :H], aa2[:, H:], exl2, h2, src, dst)

    gids = jnp.arange(G, dtype=batch.dtype)
    onehot = (batch[None, :] == gids[:, None]).astype(f32)
    counts = onehot.sum(axis=1)
    ohw = onehot / jnp.maximum(counts, 1.0)[:, None]
    ohw = jnp.zeros((G, NP), f32).at[:, :N].set(ohw)

    out = _stage_c(agg2_f, h2, exl2, den2_f, res, ohw, b2[None, :], rmat,
                   W_fc, bfc_r := b_fc[None, :])
    return (out, jnp.array(1))


# speedup vs baseline: 38.2138x; 3.8268x over previous
import functools
import jax
import jax.numpy as jnp
from jax import lax
from jax.experimental import pallas as pl
from jax.experimental.pallas import tpu as pltpu
from jax.experimental.pallas import tpu_sc as plsc

N = 10000
E = 320000
D = 128
H = 4
C = 64
HC = 256
G = 64
OUT = 128

NP = 10240
BN = 1024
NB = NP // BN


def _stage_a_body(x_ref, wcat_ref, bcat_ref, amat_ref, oh_ref, aa_ref, exl_ref):
    xb = x_ref[...]
    hres = jnp.dot(xb, wcat_ref[...], preferred_element_type=jnp.float32)
    hres = hres + bcat_ref[...]
    oh_ref[...] = hres
    aa = jnp.dot(hres[:, :HC], amat_ref[...], preferred_element_type=jnp.float32)
    aa_ref[...] = aa
    s = aa[:, :H] + aa[:, H:]
    s = jnp.where(s >= 0, s, 0.2 * s)
    exl_ref[...] = jnp.exp(s)


def _stage_a(x_p, Wcat, bcat, amat):
    return pl.pallas_call(
        _stage_a_body,
        grid=(NB,),
        in_specs=[
            pl.BlockSpec((BN, D), lambda i: (i, 0)),
            pl.BlockSpec((D, 2 * HC), lambda i: (0, 0)),
            pl.BlockSpec((1, 2 * HC), lambda i: (0, 0)),
            pl.BlockSpec((HC, 2 * H), lambda i: (0, 0)),
        ],
        out_specs=[
            pl.BlockSpec((BN, 2 * HC), lambda i: (i, 0)),
            pl.BlockSpec((BN, 2 * H), lambda i: (i, 0)),
            pl.BlockSpec((BN, H), lambda i: (i, 0)),
        ],
        out_shape=[
            jax.ShapeDtypeStruct((NP, 2 * HC), jnp.float32),
            jax.ShapeDtypeStruct((NP, 2 * H), jnp.float32),
            jax.ShapeDtypeStruct((NP, H), jnp.float32),
        ],
    )(x_p, Wcat, bcat, amat)


def _stage_b_body(agg_ref, h1_ref, exl_ref, den_ref, b1_ref, rmat_ref,
                  w2_ref, amat_ref, oh_ref, aa_ref, exl2_ref):
    exl_rep = jnp.dot(exl_ref[...], rmat_ref[...], preferred_element_type=jnp.float32)
    den_rep = jnp.dot(den_ref[...], rmat_ref[...], preferred_element_type=jnp.float32)
    num = agg_ref[...] + exl_rep * h1_ref[...]
    o1 = jnp.maximum(num / den_rep + b1_ref[...], 0.0)
    h2 = jnp.dot(o1, w2_ref[...], preferred_element_type=jnp.float32)
    oh_ref[...] = h2
    aa = jnp.dot(h2, amat_ref[...], preferred_element_type=jnp.float32)
    aa_ref[...] = aa
    s = aa[:, :H] + aa[:, H:]
    s = jnp.where(s >= 0, s, 0.2 * s)
    exl2_ref[...] = jnp.exp(s)


def _stage_b(agg, h1, exl, den, b1r, rmat, W2, amat):
    return pl.pallas_call(
        _stage_b_body,
        grid=(NB,),
        in_specs=[
            pl.BlockSpec((BN, HC), lambda i: (i, 0)),
            pl.BlockSpec((BN, HC), lambda i: (i, 0)),
            pl.BlockSpec((BN, H), lambda i: (i, 0)),
            pl.BlockSpec((BN, H), lambda i: (i, 0)),
            pl.BlockSpec((1, HC), lambda i: (0, 0)),
            pl.BlockSpec((H, HC), lambda i: (0, 0)),
            pl.BlockSpec((HC, HC), lambda i: (0, 0)),
            pl.BlockSpec((HC, 2 * H), lambda i: (0, 0)),
        ],
        out_specs=[
            pl.BlockSpec((BN, HC), lambda i: (i, 0)),
            pl.BlockSpec((BN, 2 * H), lambda i: (i, 0)),
            pl.BlockSpec((BN, H), lambda i: (i, 0)),
        ],
        out_shape=[
            jax.ShapeDtypeStruct((NP, HC), jnp.float32),
            jax.ShapeDtypeStruct((NP, 2 * H), jnp.float32),
            jax.ShapeDtypeStruct((NP, H), jnp.float32),
        ],
    )(agg, h1, exl, den, b1r, rmat, W2, amat)


def _stage_c_body(agg_ref, h2_ref, exl_ref, den_ref, res_ref, ohw_ref, b2_ref,
                  rmat_ref, wfc_ref, bfc_ref, out_ref, acc_ref):
    i = pl.program_id(0)

    @pl.when(i == 0)
    def _():
        acc_ref[...] = jnp.zeros_like(acc_ref)

    exl_rep = jnp.dot(exl_ref[...], rmat_ref[...], preferred_element_type=jnp.float32)
    den_rep = jnp.dot(den_ref[...], rmat_ref[...], preferred_element_type=jnp.float32)
    num = agg_ref[...] + exl_rep * h2_ref[...]
    hfin = jnp.maximum(num / den_rep + b2_ref[...], 0.0) + res_ref[...]
    acc_ref[...] += jnp.dot(ohw_ref[...], hfin, preferred_element_type=jnp.float32)

    @pl.when(i == NB - 1)
    def _():
        logits = jnp.dot(acc_ref[...], wfc_ref[...],
                         preferred_element_type=jnp.float32) + bfc_ref[...]
        m = jnp.max(logits, axis=1, keepdims=True)
        lse = jnp.log(jnp.sum(jnp.exp(logits - m), axis=1, keepdims=True)) + m
        out_ref[...] = logits - lse


def _stage_c(agg, h2, exl, den, res, ohw, b2r, rmat, Wfc, bfcr):
    return pl.pallas_call(
        _stage_c_body,
        grid=(NB,),
        in_specs=[
            pl.BlockSpec((BN, HC), lambda i: (i, 0)),
            pl.BlockSpec((BN, HC), lambda i: (i, 0)),
            pl.BlockSpec((BN, H), lambda i: (i, 0)),
            pl.BlockSpec((BN, H), lambda i: (i, 0)),
            pl.BlockSpec((BN, HC), lambda i: (i, 0)),
            pl.BlockSpec((G, BN), lambda i: (0, i)),
            pl.BlockSpec((1, HC), lambda i: (0, 0)),
            pl.BlockSpec((H, HC), lambda i: (0, 0)),
            pl.BlockSpec((HC, OUT), lambda i: (0, 0)),
            pl.BlockSpec((1, OUT), lambda i: (0, 0)),
        ],
        out_specs=pl.BlockSpec((G, OUT), lambda i: (0, 0)),
        out_shape=jax.ShapeDtypeStruct((G, OUT), jnp.float32),
        scratch_shapes=[pltpu.VMEM((G, HC), jnp.float32)],
    )(agg, h2, exl, den, res, ohw, b2r, rmat, Wfc, bfcr)


NT = 16
EPT = E // NT
SLN = NP // NT
CHA = 2000
CHB = 80


def _ds16(i):
    return pl.ds(pl.multiple_of(i * 16, 16), 16)


def _sc_logit_body(src_h, dst_h, als_h, ald_h, exl_h,
                   ex_o, den_o,
                   t_as0, t_ad0, t_as1, t_ad1, den_l0, den_l1,
                   src_v, dst_v, ex_v0, ex_v1, tmp_a, acc_a,
                   den_parts):
    c = lax.axis_index("c")
    s = lax.axis_index("s")
    ebase = s * EPT
    nbase = s * SLN

    def run_core(HD0, HD1):
        pltpu.sync_copy(als_h.at[pl.ds(HD0 * NP, NP)], t_as0)
        pltpu.sync_copy(ald_h.at[pl.ds(HD0 * NP, NP)], t_ad0)
        pltpu.sync_copy(als_h.at[pl.ds(HD1 * NP, NP)], t_as1)
        pltpu.sync_copy(ald_h.at[pl.ds(HD1 * NP, NP)], t_ad1)

        def zv(i, _):
            z = jnp.zeros((16,), jnp.float32)
            den_l0[_ds16(i)] = z
            den_l1[_ds16(i)] = z
            return 0
        lax.fori_loop(0, NP // 16, zv, 0)

        def chunk_a(k, _):
            pltpu.sync_copy(src_h.at[pl.ds(ebase + k * CHA, CHA)], src_v)
            pltpu.sync_copy(dst_h.at[pl.ds(ebase + k * CHA, CHA)], dst_v)

            def vec16(i, _):
                sl = _ds16(i)
                s16 = src_v[sl]
                d16 = dst_v[sl]
                for lh, (tas, tad, denl) in enumerate(
                        ((t_as0, t_ad0, den_l0), (t_as1, t_ad1, den_l1))):
                    e = (plsc.load_gather(tas, [s16])
                         + plsc.load_gather(tad, [d16]))
                    e = jnp.where(e >= 0, e, e * 0.2)
                    ex = jnp.exp(e)
                    (ex_v0 if lh == 0 else ex_v1)[_ds16(i)] = ex
                    plsc.addupdate_scatter(denl, [d16], ex)
                return 0
            lax.fori_loop(0, CHA // 16, vec16, 0)
            pltpu.sync_copy(ex_v0, ex_o.at[pl.ds(HD0 * E + ebase + k * CHA, CHA)])
            pltpu.sync_copy(ex_v1, ex_o.at[pl.ds(HD1 * E + ebase + k * CHA, CHA)])
            return 0
        lax.fori_loop(0, EPT // CHA, chunk_a, 0)

        pltpu.sync_copy(den_l0, den_parts.at[s, 0])
        pltpu.sync_copy(den_l1, den_parts.at[s, 1])
        plsc.subcore_barrier()
        for lh, HD in ((0, HD0), (1, HD1)):
            pltpu.sync_copy(exl_h.at[pl.ds(HD * NP + nbase, SLN)], acc_a)

            def red_t(t, _):
                pltpu.sync_copy(den_parts.at[t, lh, pl.ds(nbase, SLN)], tmp_a)

                def addv(i, _):
                    sl = _ds16(i)
                    acc_a[sl] = acc_a[sl] + tmp_a[sl]
                    return 0
                lax.fori_loop(0, SLN // 16, addv, 0)
                return 0
            lax.fori_loop(0, NT, red_t, 0)
            pltpu.sync_copy(acc_a, den_o.at[pl.ds(HD * NP + nbase, SLN)])

    @pl.when(c == 0)
    def _():
        run_core(0, 1)

    @pl.when(c == 1)
    def _():
        run_core(2, 3)


@functools.partial(
    pl.kernel,
    out_type=[pltpu.HBM((H * E,), jnp.float32),
              pltpu.HBM((H * NP,), jnp.float32)],
    mesh=plsc.VectorSubcoreMesh(core_axis_name="c", subcore_axis_name="s"),
    compiler_params=pltpu.CompilerParams(needs_layout_passes=False),
    scratch_types=[
        pltpu.VMEM((NP,), jnp.float32),
        pltpu.VMEM((NP,), jnp.float32),
        pltpu.VMEM((NP,), jnp.float32),
        pltpu.VMEM((NP,), jnp.float32),
        pltpu.VMEM((NP,), jnp.float32),
        pltpu.VMEM((NP,), jnp.float32),
        pltpu.VMEM((CHA,), jnp.int32),
        pltpu.VMEM((CHA,), jnp.int32),
        pltpu.VMEM((CHA,), jnp.float32),
        pltpu.VMEM((CHA,), jnp.float32),
        pltpu.VMEM((SLN,), jnp.float32),
        pltpu.VMEM((SLN,), jnp.float32),
        pltpu.VMEM_SHARED((NT, 2, NP), jnp.float32),
    ],
)
def _sc_logit_kernel(*refs):
    _sc_logit_body(*refs)


def _sc_agg_body(src_h, dst_h, ex_h, hA, hB, zrow_h,
                 aggA, aggB,
                 srcB, dstB, ex_v0, ex_v1, rows,
                 acc_sp, sem):
    c = lax.axis_index("c")
    s = lax.axis_index("s")
    ebase = s * EPT
    nbase = s * SLN

    def run_core(h_t, agg_t, HD0, HD1):
        pltpu.sync_copy(zrow_h, acc_sp.at[pl.ds(nbase, SLN)])
        plsc.subcore_barrier()

        def chunk_b(k, _):
            eb = ebase + k * CHB
            pltpu.sync_copy(src_h.at[pl.ds(eb, CHB)], srcB)
            pltpu.sync_copy(dst_h.at[pl.ds(eb, CHB)], dstB)
            pltpu.sync_copy(ex_h.at[pl.ds(HD0 * E + eb, CHB)], ex_v0)
            pltpu.sync_copy(ex_h.at[pl.ds(HD1 * E + eb, CHB)], ex_v1)
            pltpu.async_copy(h_t.at[srcB], rows, sem).wait()

            def scale16(g, _):
                off = _ds16(g)
                ex16a = ex_v0[off]
                ex16b = ex_v1[off]
                for j in range(16):
                    ea = ex16a[j]
                    eb_ = ex16b[j]
                    r = g * 16 + j
                    for jj in range(C // 16):
                        rows[r, _ds16(jj)] = rows[r, _ds16(jj)] * ea
                    for jj in range(C // 16):
                        sl = pl.ds(pl.multiple_of(C + jj * 16, 16), 16)
                        rows[r, sl] = rows[r, sl] * eb_
                return 0
            lax.fori_loop(0, CHB // 16, scale16, 0)
            pltpu.sync_copy(rows, acc_sp.at[dstB], add=True)
            return 0
        lax.fori_loop(0, EPT // CHB, chunk_b, 0)

        plsc.subcore_barrier()
        pltpu.sync_copy(acc_sp.at[pl.ds(nbase, SLN)],
                        agg_t.at[pl.ds(nbase, SLN)])

    @pl.when(c == 0)
    def _():
        run_core(hA, aggA, 0, 1)

    @pl.when(c == 1)
    def _():
        run_core(hB, aggB, 2, 3)


@functools.partial(
    pl.kernel,
    out_type=[pltpu.HBM((NP, 2 * C), jnp.float32),
              pltpu.HBM((NP, 2 * C), jnp.float32)],
    mesh=plsc.VectorSubcoreMesh(core_axis_name="c", subcore_axis_name="s"),
    compiler_params=pltpu.CompilerParams(needs_layout_passes=False),
    scratch_types=[
        pltpu.VMEM((CHB,), jnp.int32),
        pltpu.VMEM((CHB,), jnp.int32),
        pltpu.VMEM((CHB,), jnp.float32),
        pltpu.VMEM((CHB,), jnp.float32),
        pltpu.VMEM((CHB, 2 * C), jnp.float32),
        pltpu.VMEM_SHARED((NP, 2 * C), jnp.float32),
        pltpu.SemaphoreType.DMA,
    ],
)
def _sc_agg_kernel(*refs):
    _sc_agg_body(*refs)


def _gat_edge_sc(als, ald, exl, h_heads, src, dst):
    alsT = als.T.reshape(-1)
    aldT = ald.T.reshape(-1)
    exlT = exl.T.reshape(-1)
    ex, den = _sc_logit_kernel(src, dst, alsT, aldT, exlT)
    zrow = jnp.zeros((SLN, 2 * C), jnp.float32)
    aggA, aggB = _sc_agg_kernel(src, dst, ex,
                                h_heads[:, :2 * C], h_heads[:, 2 * C:], zrow)
    agg = jnp.concatenate([aggA, aggB], axis=1)
    return agg, den.reshape(H, NP).T


def kernel(x, edge_index, batch, W1, a_src1, a_dst1, b1, W2, a_src2, a_dst2,
           b2, W_res, b_res, W_fc, b_fc):
    f32 = jnp.float32
    x_p = jnp.zeros((NP, D), f32).at[:N].set(x)
    src = edge_index[0]
    dst = edge_index[1]

    eyeC = jnp.eye(H, dtype=f32)
    amat1 = jnp.concatenate(
        [jnp.einsum('hc,hk->hck', a_src1, eyeC).reshape(HC, H),
         jnp.einsum('hc,hk->hck', a_dst1, eyeC).reshape(HC, H)], axis=1)
    amat2 = jnp.concatenate(
        [jnp.einsum('hc,hk->hck', a_src2, eyeC).reshape(HC, H),
         jnp.einsum('hc,hk->hck', a_dst2, eyeC).reshape(HC, H)], axis=1)
    rmat = jnp.repeat(jnp.eye(H, dtype=f32), C, axis=1)

    Wcat = jnp.concatenate([W1, W_res], axis=1)
    bcat = jnp.concatenate([jnp.zeros((HC,), f32), b_res])[None, :]

    oh, aa1, exl1 = _stage_a(x_p, Wcat, bcat, amat1)
    h1 = oh[:, :HC]
    res = oh[:, HC:]

    agg1_f, den1_f = _gat_edge_sc(aa1[:, :H], aa1[:, H:], exl1, h1, src, dst)

    h2, aa2, exl2 = _stage_b(agg1_f, h1, exl1, den1_f, b1[None, :], rmat,
                             W2, amat2)

    agg2_f, den2_f = _gat_edge_sc(aa2[:, :H], aa2[:, H:], exl2, h2, src, dst)

    gids = jnp.arange(G, dtype=batch.dtype)
    onehot = (batch[None, :] == gids[:, None]).astype(f32)
    counts = onehot.sum(axis=1)
    ohw = onehot / jnp.maximum(counts, 1.0)[:, None]
    ohw = jnp.zeros((G, NP), f32).at[:, :N].set(ohw)

    out = _stage_c(agg2_f, h2, exl2, den2_f, res, ohw, b2[None, :], rmat,
                   W_fc, bfc_r := b_fc[None, :])
    return (out, jnp.array(1))


# speedup vs baseline: 50.0942x; 1.3109x over previous
import functools
import jax
import jax.numpy as jnp
from jax import lax
from jax.experimental import pallas as pl
from jax.experimental.pallas import tpu as pltpu
from jax.experimental.pallas import tpu_sc as plsc

N = 10000
E = 320000
D = 128
H = 4
C = 64
HC = 256
G = 64
OUT = 128

NP = 10240
BN = 1024
NB = NP // BN


def _stage_a_body(x_ref, wcat_ref, bcat_ref, amat_ref, oh_ref, aa_ref, exl_ref):
    xb = x_ref[...]
    hres = jnp.dot(xb, wcat_ref[...], preferred_element_type=jnp.float32)
    hres = hres + bcat_ref[...]
    oh_ref[...] = hres
    aa = jnp.dot(hres[:, :HC], amat_ref[...], preferred_element_type=jnp.float32)
    aa_ref[...] = aa
    s = aa[:, :H] + aa[:, H:]
    s = jnp.where(s >= 0, s, 0.2 * s)
    exl_ref[...] = jnp.exp(s)


def _stage_a(x_p, Wcat, bcat, amat):
    return pl.pallas_call(
        _stage_a_body,
        grid=(NB,),
        in_specs=[
            pl.BlockSpec((BN, D), lambda i: (i, 0)),
            pl.BlockSpec((D, 2 * HC), lambda i: (0, 0)),
            pl.BlockSpec((1, 2 * HC), lambda i: (0, 0)),
            pl.BlockSpec((HC, 2 * H), lambda i: (0, 0)),
        ],
        out_specs=[
            pl.BlockSpec((BN, 2 * HC), lambda i: (i, 0)),
            pl.BlockSpec((BN, 2 * H), lambda i: (i, 0)),
            pl.BlockSpec((BN, H), lambda i: (i, 0)),
        ],
        out_shape=[
            jax.ShapeDtypeStruct((NP, 2 * HC), jnp.float32),
            jax.ShapeDtypeStruct((NP, 2 * H), jnp.float32),
            jax.ShapeDtypeStruct((NP, H), jnp.float32),
        ],
    )(x_p, Wcat, bcat, amat)


def _stage_b_body(agg_ref, h1_ref, exl_ref, den_ref, b1_ref, rmat_ref,
                  w2_ref, amat_ref, oh_ref, aa_ref, exl2_ref):
    exl_rep = jnp.dot(exl_ref[...], rmat_ref[...], preferred_element_type=jnp.float32)
    den_rep = jnp.dot(den_ref[...], rmat_ref[...], preferred_element_type=jnp.float32)
    num = agg_ref[...] + exl_rep * h1_ref[...]
    o1 = jnp.maximum(num / den_rep + b1_ref[...], 0.0)
    h2 = jnp.dot(o1, w2_ref[...], preferred_element_type=jnp.float32)
    oh_ref[...] = h2
    aa = jnp.dot(h2, amat_ref[...], preferred_element_type=jnp.float32)
    aa_ref[...] = aa
    s = aa[:, :H] + aa[:, H:]
    s = jnp.where(s >= 0, s, 0.2 * s)
    exl2_ref[...] = jnp.exp(s)


def _stage_b(agg, h1, exl, den, b1r, rmat, W2, amat):
    return pl.pallas_call(
        _stage_b_body,
        grid=(NB,),
        in_specs=[
            pl.BlockSpec((BN, HC), lambda i: (i, 0)),
            pl.BlockSpec((BN, HC), lambda i: (i, 0)),
            pl.BlockSpec((BN, H), lambda i: (i, 0)),
            pl.BlockSpec((BN, H), lambda i: (i, 0)),
            pl.BlockSpec((1, HC), lambda i: (0, 0)),
            pl.BlockSpec((H, HC), lambda i: (0, 0)),
            pl.BlockSpec((HC, HC), lambda i: (0, 0)),
            pl.BlockSpec((HC, 2 * H), lambda i: (0, 0)),
        ],
        out_specs=[
            pl.BlockSpec((BN, HC), lambda i: (i, 0)),
            pl.BlockSpec((BN, 2 * H), lambda i: (i, 0)),
            pl.BlockSpec((BN, H), lambda i: (i, 0)),
        ],
        out_shape=[
            jax.ShapeDtypeStruct((NP, HC), jnp.float32),
            jax.ShapeDtypeStruct((NP, 2 * H), jnp.float32),
            jax.ShapeDtypeStruct((NP, H), jnp.float32),
        ],
    )(agg, h1, exl, den, b1r, rmat, W2, amat)


def _stage_c_body(agg_ref, h2_ref, exl_ref, den_ref, res_ref, ohw_ref, b2_ref,
                  rmat_ref, wfc_ref, bfc_ref, out_ref, acc_ref):
    i = pl.program_id(0)

    @pl.when(i == 0)
    def _():
        acc_ref[...] = jnp.zeros_like(acc_ref)

    exl_rep = jnp.dot(exl_ref[...], rmat_ref[...], preferred_element_type=jnp.float32)
    den_rep = jnp.dot(den_ref[...], rmat_ref[...], preferred_element_type=jnp.float32)
    num = agg_ref[...] + exl_rep * h2_ref[...]
    hfin = jnp.maximum(num / den_rep + b2_ref[...], 0.0) + res_ref[...]
    acc_ref[...] += jnp.dot(ohw_ref[...], hfin, preferred_element_type=jnp.float32)

    @pl.when(i == NB - 1)
    def _():
        logits = jnp.dot(acc_ref[...], wfc_ref[...],
                         preferred_element_type=jnp.float32) + bfc_ref[...]
        m = jnp.max(logits, axis=1, keepdims=True)
        lse = jnp.log(jnp.sum(jnp.exp(logits - m), axis=1, keepdims=True)) + m
        out_ref[...] = logits - lse


def _stage_c(agg, h2, exl, den, res, ohw, b2r, rmat, Wfc, bfcr):
    return pl.pallas_call(
        _stage_c_body,
        grid=(NB,),
        in_specs=[
            pl.BlockSpec((BN, HC), lambda i: (i, 0)),
            pl.BlockSpec((BN, HC), lambda i: (i, 0)),
            pl.BlockSpec((BN, H), lambda i: (i, 0)),
            pl.BlockSpec((BN, H), lambda i: (i, 0)),
            pl.BlockSpec((BN, HC), lambda i: (i, 0)),
            pl.BlockSpec((G, BN), lambda i: (0, i)),
            pl.BlockSpec((1, HC), lambda i: (0, 0)),
            pl.BlockSpec((H, HC), lambda i: (0, 0)),
            pl.BlockSpec((HC, OUT), lambda i: (0, 0)),
            pl.BlockSpec((1, OUT), lambda i: (0, 0)),
        ],
        out_specs=pl.BlockSpec((G, OUT), lambda i: (0, 0)),
        out_shape=jax.ShapeDtypeStruct((G, OUT), jnp.float32),
        scratch_shapes=[pltpu.VMEM((G, HC), jnp.float32)],
    )(agg, h2, exl, den, res, ohw, b2r, rmat, Wfc, bfcr)


NT = 16
EPT = E // NT
SLN = NP // NT
CHA = 2000
CHB = 80


def _ds16(i):
    return pl.ds(pl.multiple_of(i * 16, 16), 16)


def _sc_logit_body(src_h, dst_h, als_h, ald_h, exl_h,
                   ex_o, den_o,
                   t_as0, t_ad0, t_as1, t_ad1, den_l0, den_l1,
                   src_v, dst_v, ex_v0, ex_v1, tmp_a, acc_a,
                   den_parts):
    c = lax.axis_index("c")
    s = lax.axis_index("s")
    ebase = s * EPT
    nbase = s * SLN

    def run_core(HD0, HD1):
        pltpu.sync_copy(als_h.at[pl.ds(HD0 * NP, NP)], t_as0)
        pltpu.sync_copy(ald_h.at[pl.ds(HD0 * NP, NP)], t_ad0)
        pltpu.sync_copy(als_h.at[pl.ds(HD1 * NP, NP)], t_as1)
        pltpu.sync_copy(ald_h.at[pl.ds(HD1 * NP, NP)], t_ad1)

        def zv(i, _):
            z = jnp.zeros((16,), jnp.float32)
            den_l0[_ds16(i)] = z
            den_l1[_ds16(i)] = z
            return 0
        lax.fori_loop(0, NP // 16, zv, 0)

        def chunk_a(k, _):
            pltpu.sync_copy(src_h.at[pl.ds(ebase + k * CHA, CHA)], src_v)
            pltpu.sync_copy(dst_h.at[pl.ds(ebase + k * CHA, CHA)], dst_v)

            def vec16(i, _):
                sl = _ds16(i)
                s16 = src_v[sl]
                d16 = dst_v[sl]
                for lh, (tas, tad, denl) in enumerate(
                        ((t_as0, t_ad0, den_l0), (t_as1, t_ad1, den_l1))):
                    e = (plsc.load_gather(tas, [s16])
                         + plsc.load_gather(tad, [d16]))
                    e = jnp.where(e >= 0, e, e * 0.2)
                    ex = jnp.exp(e)
                    (ex_v0 if lh == 0 else ex_v1)[_ds16(i)] = ex
                    plsc.addupdate_scatter(denl, [d16], ex)
                return 0
            lax.fori_loop(0, CHA // 16, vec16, 0)
            pltpu.sync_copy(ex_v0, ex_o.at[pl.ds(HD0 * E + ebase + k * CHA, CHA)])
            pltpu.sync_copy(ex_v1, ex_o.at[pl.ds(HD1 * E + ebase + k * CHA, CHA)])
            return 0
        lax.fori_loop(0, EPT // CHA, chunk_a, 0)

        pltpu.sync_copy(den_l0, den_parts.at[s, 0])
        pltpu.sync_copy(den_l1, den_parts.at[s, 1])
        plsc.subcore_barrier()
        for lh, HD in ((0, HD0), (1, HD1)):
            pltpu.sync_copy(exl_h.at[pl.ds(HD * NP + nbase, SLN)], acc_a)

            def red_t(t, _):
                pltpu.sync_copy(den_parts.at[t, lh, pl.ds(nbase, SLN)], tmp_a)

                def addv(i, _):
                    sl = _ds16(i)
                    acc_a[sl] = acc_a[sl] + tmp_a[sl]
                    return 0
                lax.fori_loop(0, SLN // 16, addv, 0)
                return 0
            lax.fori_loop(0, NT, red_t, 0)
            pltpu.sync_copy(acc_a, den_o.at[pl.ds(HD * NP + nbase, SLN)])

    @pl.when(c == 0)
    def _():
        run_core(0, 1)

    @pl.when(c == 1)
    def _():
        run_core(2, 3)


@functools.partial(
    pl.kernel,
    out_type=[pltpu.HBM((H * E,), jnp.float32),
              pltpu.HBM((H * NP,), jnp.float32)],
    mesh=plsc.VectorSubcoreMesh(core_axis_name="c", subcore_axis_name="s"),
    compiler_params=pltpu.CompilerParams(needs_layout_passes=False),
    scratch_types=[
        pltpu.VMEM((NP,), jnp.float32),
        pltpu.VMEM((NP,), jnp.float32),
        pltpu.VMEM((NP,), jnp.float32),
        pltpu.VMEM((NP,), jnp.float32),
        pltpu.VMEM((NP,), jnp.float32),
        pltpu.VMEM((NP,), jnp.float32),
        pltpu.VMEM((CHA,), jnp.int32),
        pltpu.VMEM((CHA,), jnp.int32),
        pltpu.VMEM((CHA,), jnp.float32),
        pltpu.VMEM((CHA,), jnp.float32),
        pltpu.VMEM((SLN,), jnp.float32),
        pltpu.VMEM((SLN,), jnp.float32),
        pltpu.VMEM_SHARED((NT, 2, NP), jnp.float32),
    ],
)
def _sc_logit_kernel(*refs):
    _sc_logit_body(*refs)


def _sc_agg_body(src_h, dst_h, ex_h, hA, hB, zrow_h,
                 aggA, aggB,
                 srcB0, dstB0, exa0, exb0, rows0,
                 srcB1, dstB1, exa1, exb1, rows1,
                 acc_sp, sem0, sem1):
    c = lax.axis_index("c")
    s = lax.axis_index("s")
    ebase = s * EPT
    nbase = s * SLN
    NC2 = (EPT // CHB) // 2

    def run_core(h_t, agg_t, HD0, HD1):
        pltpu.sync_copy(zrow_h, acc_sp.at[pl.ds(nbase, SLN)])
        plsc.subcore_barrier()

        bufs = ((srcB0, dstB0, exa0, exb0, rows0, sem0),
                (srcB1, dstB1, exa1, exb1, rows1, sem1))

        def load_idx(k, b):
            eb = ebase + k * CHB
            srcB, _dstB, exa, exb, rows, sem = bufs[b]
            pltpu.sync_copy(src_h.at[pl.ds(eb, CHB)], srcB)
            pltpu.sync_copy(dst_h.at[pl.ds(eb, CHB)], _dstB)
            pltpu.sync_copy(ex_h.at[pl.ds(HD0 * E + eb, CHB)], exa)
            pltpu.sync_copy(ex_h.at[pl.ds(HD1 * E + eb, CHB)], exb)
            pltpu.async_copy(h_t.at[srcB], rows, sem)

        def process(b):
            srcB, _dstB, exa, exb, rows, sem = bufs[b]
            pltpu.make_async_copy(h_t.at[srcB], rows, sem).wait()

            def scale16(g, _):
                off = _ds16(g)
                ex16a = exa[off]
                ex16b = exb[off]
                for j in range(16):
                    ea = ex16a[j]
                    eb_ = ex16b[j]
                    r = g * 16 + j
                    for jj in range(C // 16):
                        rows[r, _ds16(jj)] = rows[r, _ds16(jj)] * ea
                    for jj in range(C // 16):
                        sl = pl.ds(pl.multiple_of(C + jj * 16, 16), 16)
                        rows[r, sl] = rows[r, sl] * eb_
                return 0
            lax.fori_loop(0, CHB // 16, scale16, 0)
            pltpu.sync_copy(rows, acc_sp.at[_dstB], add=True)

        load_idx(0, 0)

        def outer(ko, _):
            load_idx(2 * ko + 1, 1)
            process(0)

            @pl.when(ko < NC2 - 1)
            def _():
                load_idx(2 * ko + 2, 0)
            process(1)
            return 0
        lax.fori_loop(0, NC2, outer, 0)

        plsc.subcore_barrier()
        pltpu.sync_copy(acc_sp.at[pl.ds(nbase, SLN)],
                        agg_t.at[pl.ds(nbase, SLN)])

    @pl.when(c == 0)
    def _():
        run_core(hA, aggA, 0, 1)

    @pl.when(c == 1)
    def _():
        run_core(hB, aggB, 2, 3)


@functools.partial(
    pl.kernel,
    out_type=[pltpu.HBM((NP, 2 * C), jnp.float32),
              pltpu.HBM((NP, 2 * C), jnp.float32)],
    mesh=plsc.VectorSubcoreMesh(core_axis_name="c", subcore_axis_name="s"),
    compiler_params=pltpu.CompilerParams(needs_layout_passes=False),
    scratch_types=[
        pltpu.VMEM((CHB,), jnp.int32),
        pltpu.VMEM((CHB,), jnp.int32),
        pltpu.VMEM((CHB,), jnp.float32),
        pltpu.VMEM((CHB,), jnp.float32),
        pltpu.VMEM((CHB, 2 * C), jnp.float32),
        pltpu.VMEM((CHB,), jnp.int32),
        pltpu.VMEM((CHB,), jnp.int32),
        pltpu.VMEM((CHB,), jnp.float32),
        pltpu.VMEM((CHB,), jnp.float32),
        pltpu.VMEM((CHB, 2 * C), jnp.float32),
        pltpu.VMEM_SHARED((NP, 2 * C), jnp.float32),
        pltpu.SemaphoreType.DMA,
        pltpu.SemaphoreType.DMA,
    ],
)
def _sc_agg_kernel(*refs):
    _sc_agg_body(*refs)


def _gat_edge_sc(als, ald, exl, h_heads, src, dst):
    alsT = als.T.reshape(-1)
    aldT = ald.T.reshape(-1)
    exlT = exl.T.reshape(-1)
    ex, den = _sc_logit_kernel(src, dst, alsT, aldT, exlT)
    zrow = jnp.zeros((SLN, 2 * C), jnp.float32)
    aggA, aggB = _sc_agg_kernel(src, dst, ex,
                                h_heads[:, :2 * C], h_heads[:, 2 * C:], zrow)
    agg = jnp.concatenate([aggA, aggB], axis=1)
    return agg, den.reshape(H, NP).T


def kernel(x, edge_index, batch, W1, a_src1, a_dst1, b1, W2, a_src2, a_dst2,
           b2, W_res, b_res, W_fc, b_fc):
    f32 = jnp.float32
    x_p = jnp.zeros((NP, D), f32).at[:N].set(x)
    src = edge_index[0]
    dst = edge_index[1]

    eyeC = jnp.eye(H, dtype=f32)
    amat1 = jnp.concatenate(
        [jnp.einsum('hc,hk->hck', a_src1, eyeC).reshape(HC, H),
         jnp.einsum('hc,hk->hck', a_dst1, eyeC).reshape(HC, H)], axis=1)
    amat2 = jnp.concatenate(
        [jnp.einsum('hc,hk->hck', a_src2, eyeC).reshape(HC, H),
         jnp.einsum('hc,hk->hck', a_dst2, eyeC).reshape(HC, H)], axis=1)
    rmat = jnp.repeat(jnp.eye(H, dtype=f32), C, axis=1)

    Wcat = jnp.concatenate([W1, W_res], axis=1)
    bcat = jnp.concatenate([jnp.zeros((HC,), f32), b_res])[None, :]

    oh, aa1, exl1 = _stage_a(x_p, Wcat, bcat, amat1)
    h1 = oh[:, :HC]
    res = oh[:, HC:]

    agg1_f, den1_f = _gat_edge_sc(aa1[:, :H], aa1[:, H:], exl1, h1, src, dst)

    h2, aa2, exl2 = _stage_b(agg1_f, h1, exl1, den1_f, b1[None, :], rmat,
                             W2, amat2)

    agg2_f, den2_f = _gat_edge_sc(aa2[:, :H], aa2[:, H:], exl2, h2, src, dst)

    gids = jnp.arange(G, dtype=batch.dtype)
    onehot = (batch[None, :] == gids[:, None]).astype(f32)
    counts = onehot.sum(axis=1)
    ohw = onehot / jnp.maximum(counts, 1.0)[:, None]
    ohw = jnp.zeros((G, NP), f32).at[:, :N].set(ohw)

    out = _stage_c(agg2_f, h2, exl2, den2_f, res, ohw, b2[None, :], rmat,
                   W_fc, bfc_r := b_fc[None, :])
    return (out, jnp.array(1))


# speedup vs baseline: 77.6920x; 1.5509x over previous
import functools
import jax
import jax.numpy as jnp
from jax import lax
from jax.experimental import pallas as pl
from jax.experimental.pallas import tpu as pltpu
from jax.experimental.pallas import tpu_sc as plsc

N = 10000
E = 320000
D = 128
H = 4
C = 64
HC = 256
G = 64
OUT = 128

NP = 10240
BN = 1024
NB = NP // BN


def _stage_a_body(x_ref, wcat_ref, bcat_ref, amat_ref, oh_ref, aa_ref, exl_ref):
    xb = x_ref[...]
    hres = jnp.dot(xb, wcat_ref[...], preferred_element_type=jnp.float32)
    hres = hres + bcat_ref[...]
    oh_ref[...] = hres
    aa = jnp.dot(hres[:, :HC], amat_ref[...], preferred_element_type=jnp.float32)
    aa_ref[...] = aa
    s = aa[:, :H] + aa[:, H:]
    s = jnp.where(s >= 0, s, 0.2 * s)
    exl_ref[...] = jnp.exp(s)


def _stage_a(x_p, Wcat, bcat, amat):
    return pl.pallas_call(
        _stage_a_body,
        grid=(NB,),
        in_specs=[
            pl.BlockSpec((BN, D), lambda i: (i, 0)),
            pl.BlockSpec((D, 2 * HC), lambda i: (0, 0)),
            pl.BlockSpec((1, 2 * HC), lambda i: (0, 0)),
            pl.BlockSpec((HC, 2 * H), lambda i: (0, 0)),
        ],
        out_specs=[
            pl.BlockSpec((BN, 2 * HC), lambda i: (i, 0)),
            pl.BlockSpec((BN, 2 * H), lambda i: (i, 0)),
            pl.BlockSpec((BN, H), lambda i: (i, 0)),
        ],
        out_shape=[
            jax.ShapeDtypeStruct((NP, 2 * HC), jnp.float32),
            jax.ShapeDtypeStruct((NP, 2 * H), jnp.float32),
            jax.ShapeDtypeStruct((NP, H), jnp.float32),
        ],
    )(x_p, Wcat, bcat, amat)


def _stage_b_body(agg_ref, h1_ref, exl_ref, den_ref, b1_ref, rmat_ref,
                  w2_ref, amat_ref, oh_ref, aa_ref, exl2_ref):
    exl_rep = jnp.dot(exl_ref[...], rmat_ref[...], preferred_element_type=jnp.float32)
    den_rep = jnp.dot(den_ref[...], rmat_ref[...], preferred_element_type=jnp.float32)
    num = agg_ref[...] + exl_rep * h1_ref[...]
    o1 = jnp.maximum(num / den_rep + b1_ref[...], 0.0)
    h2 = jnp.dot(o1, w2_ref[...], preferred_element_type=jnp.float32)
    oh_ref[...] = h2
    aa = jnp.dot(h2, amat_ref[...], preferred_element_type=jnp.float32)
    aa_ref[...] = aa
    s = aa[:, :H] + aa[:, H:]
    s = jnp.where(s >= 0, s, 0.2 * s)
    exl2_ref[...] = jnp.exp(s)


def _stage_b(agg, h1, exl, den, b1r, rmat, W2, amat):
    return pl.pallas_call(
        _stage_b_body,
        grid=(NB,),
        in_specs=[
            pl.BlockSpec((BN, HC), lambda i: (i, 0)),
            pl.BlockSpec((BN, HC), lambda i: (i, 0)),
            pl.BlockSpec((BN, H), lambda i: (i, 0)),
            pl.BlockSpec((BN, H), lambda i: (i, 0)),
            pl.BlockSpec((1, HC), lambda i: (0, 0)),
            pl.BlockSpec((H, HC), lambda i: (0, 0)),
            pl.BlockSpec((HC, HC), lambda i: (0, 0)),
            pl.BlockSpec((HC, 2 * H), lambda i: (0, 0)),
        ],
        out_specs=[
            pl.BlockSpec((BN, HC), lambda i: (i, 0)),
            pl.BlockSpec((BN, 2 * H), lambda i: (i, 0)),
            pl.BlockSpec((BN, H), lambda i: (i, 0)),
        ],
        out_shape=[
            jax.ShapeDtypeStruct((NP, HC), jnp.float32),
            jax.ShapeDtypeStruct((NP, 2 * H), jnp.float32),
            jax.ShapeDtypeStruct((NP, H), jnp.float32),
        ],
    )(agg, h1, exl, den, b1r, rmat, W2, amat)


def _stage_c_body(agg_ref, h2_ref, exl_ref, den_ref, res_ref, ohw_ref, b2_ref,
                  rmat_ref, wfc_ref, bfc_ref, out_ref, acc_ref):
    i = pl.program_id(0)

    @pl.when(i == 0)
    def _():
        acc_ref[...] = jnp.zeros_like(acc_ref)

    exl_rep = jnp.dot(exl_ref[...], rmat_ref[...], preferred_element_type=jnp.float32)
    den_rep = jnp.dot(den_ref[...], rmat_ref[...], preferred_element_type=jnp.float32)
    num = agg_ref[...] + exl_rep * h2_ref[...]
    hfin = jnp.maximum(num / den_rep + b2_ref[...], 0.0) + res_ref[...]
    acc_ref[...] += jnp.dot(ohw_ref[...], hfin, preferred_element_type=jnp.float32)

    @pl.when(i == NB - 1)
    def _():
        logits = jnp.dot(acc_ref[...], wfc_ref[...],
                         preferred_element_type=jnp.float32) + bfc_ref[...]
        m = jnp.max(logits, axis=1, keepdims=True)
        lse = jnp.log(jnp.sum(jnp.exp(logits - m), axis=1, keepdims=True)) + m
        out_ref[...] = logits - lse


def _stage_c(agg, h2, exl, den, res, ohw, b2r, rmat, Wfc, bfcr):
    return pl.pallas_call(
        _stage_c_body,
        grid=(NB,),
        in_specs=[
            pl.BlockSpec((BN, HC), lambda i: (i, 0)),
            pl.BlockSpec((BN, HC), lambda i: (i, 0)),
            pl.BlockSpec((BN, H), lambda i: (i, 0)),
            pl.BlockSpec((BN, H), lambda i: (i, 0)),
            pl.BlockSpec((BN, HC), lambda i: (i, 0)),
            pl.BlockSpec((G, BN), lambda i: (0, i)),
            pl.BlockSpec((1, HC), lambda i: (0, 0)),
            pl.BlockSpec((H, HC), lambda i: (0, 0)),
            pl.BlockSpec((HC, OUT), lambda i: (0, 0)),
            pl.BlockSpec((1, OUT), lambda i: (0, 0)),
        ],
        out_specs=pl.BlockSpec((G, OUT), lambda i: (0, 0)),
        out_shape=jax.ShapeDtypeStruct((G, OUT), jnp.float32),
        scratch_shapes=[pltpu.VMEM((G, HC), jnp.float32)],
    )(agg, h2, exl, den, res, ohw, b2r, rmat, Wfc, bfcr)


NT = 16
EPT = E // NT
SLN = NP // NT
CHA = 2000
CHB = 80


def _ds16(i):
    return pl.ds(pl.multiple_of(i * 16, 16), 16)


def _sc_logit_body(src_h, dst_h, als_h, ald_h, exl_h,
                   ex_o, den_o,
                   t_as0, t_ad0, t_as1, t_ad1, den_l0, den_l1,
                   src_v, dst_v, ex_v0, ex_v1, tmp_a, acc_a,
                   den_parts):
    c = lax.axis_index("c")
    s = lax.axis_index("s")
    ebase = s * EPT
    nbase = s * SLN

    def run_core(HD0, HD1):
        pltpu.sync_copy(als_h.at[pl.ds(HD0 * NP, NP)], t_as0)
        pltpu.sync_copy(ald_h.at[pl.ds(HD0 * NP, NP)], t_ad0)
        pltpu.sync_copy(als_h.at[pl.ds(HD1 * NP, NP)], t_as1)
        pltpu.sync_copy(ald_h.at[pl.ds(HD1 * NP, NP)], t_ad1)

        def zv(i, _):
            z = jnp.zeros((16,), jnp.float32)
            den_l0[_ds16(i)] = z
            den_l1[_ds16(i)] = z
            return 0
        lax.fori_loop(0, NP // 16, zv, 0)

        def chunk_a(k, _):
            pltpu.sync_copy(src_h.at[pl.ds(ebase + k * CHA, CHA)], src_v)
            pltpu.sync_copy(dst_h.at[pl.ds(ebase + k * CHA, CHA)], dst_v)

            def vec16(i, _):
                sl = _ds16(i)
                s16 = src_v[sl]
                d16 = dst_v[sl]
                for lh, (tas, tad, denl) in enumerate(
                        ((t_as0, t_ad0, den_l0), (t_as1, t_ad1, den_l1))):
                    e = (plsc.load_gather(tas, [s16])
                         + plsc.load_gather(tad, [d16]))
                    e = jnp.where(e >= 0, e, e * 0.2)
                    ex = jnp.exp(e)
                    (ex_v0 if lh == 0 else ex_v1)[_ds16(i)] = ex
                    plsc.addupdate_scatter(denl, [d16], ex)
                return 0
            lax.fori_loop(0, CHA // 16, vec16, 0)
            pltpu.sync_copy(ex_v0, ex_o.at[pl.ds(HD0 * E + ebase + k * CHA, CHA)])
            pltpu.sync_copy(ex_v1, ex_o.at[pl.ds(HD1 * E + ebase + k * CHA, CHA)])
            return 0
        lax.fori_loop(0, EPT // CHA, chunk_a, 0)

        pltpu.sync_copy(den_l0, den_parts.at[s, 0])
        pltpu.sync_copy(den_l1, den_parts.at[s, 1])
        plsc.subcore_barrier()
        for lh, HD in ((0, HD0), (1, HD1)):
            pltpu.sync_copy(exl_h.at[pl.ds(HD * NP + nbase, SLN)], acc_a)

            def red_t(t, _):
                pltpu.sync_copy(den_parts.at[t, lh, pl.ds(nbase, SLN)], tmp_a)

                def addv(i, _):
                    sl = _ds16(i)
                    acc_a[sl] = acc_a[sl] + tmp_a[sl]
                    return 0
                lax.fori_loop(0, SLN // 16, addv, 0)
                return 0
            lax.fori_loop(0, NT, red_t, 0)
            pltpu.sync_copy(acc_a, den_o.at[pl.ds(HD * NP + nbase, SLN)])

    @pl.when(c == 0)
    def _():
        run_core(0, 1)

    @pl.when(c == 1)
    def _():
        run_core(2, 3)


@functools.partial(
    pl.kernel,
    out_type=[pltpu.HBM((H * E,), jnp.float32),
              pltpu.HBM((H * NP,), jnp.float32)],
    mesh=plsc.VectorSubcoreMesh(core_axis_name="c", subcore_axis_name="s"),
    compiler_params=pltpu.CompilerParams(needs_layout_passes=False),
    scratch_types=[
        pltpu.VMEM((NP,), jnp.float32),
        pltpu.VMEM((NP,), jnp.float32),
        pltpu.VMEM((NP,), jnp.float32),
        pltpu.VMEM((NP,), jnp.float32),
        pltpu.VMEM((NP,), jnp.float32),
        pltpu.VMEM((NP,), jnp.float32),
        pltpu.VMEM((CHA,), jnp.int32),
        pltpu.VMEM((CHA,), jnp.int32),
        pltpu.VMEM((CHA,), jnp.float32),
        pltpu.VMEM((CHA,), jnp.float32),
        pltpu.VMEM((SLN,), jnp.float32),
        pltpu.VMEM((SLN,), jnp.float32),
        pltpu.VMEM_SHARED((NT, 2, NP), jnp.float32),
    ],
)
def _sc_logit_kernel(*refs):
    _sc_logit_body(*refs)


def _sc_agg_body(src_h, dst_h, ex_h, hA, hB, zrow_h,
                 aggA, aggB,
                 srcB0, dstB0, exa0, exb0, rows0,
                 srcB1, dstB1, exa1, exb1, rows1,
                 acc_sp, semG0, semG1, semI0, semI1):
    c = lax.axis_index("c")
    s = lax.axis_index("s")
    ebase = s * EPT
    nbase = s * SLN
    NC2 = (EPT // CHB) // 2

    def run_core(h_t, agg_t, HD0, HD1):
        pltpu.sync_copy(zrow_h, acc_sp.at[pl.ds(nbase, SLN)])
        plsc.subcore_barrier()

        bufs = ((srcB0, dstB0, exa0, exb0, rows0, semG0, semI0),
                (srcB1, dstB1, exa1, exb1, rows1, semG1, semI1))

        def idx_copies(k, b):
            eb = ebase + k * CHB
            srcB, dstB, exa, exb, _rows, _sG, sI = bufs[b]
            return (pltpu.make_async_copy(src_h.at[pl.ds(eb, CHB)], srcB, sI),
                    pltpu.make_async_copy(dst_h.at[pl.ds(eb, CHB)], dstB, sI),
                    pltpu.make_async_copy(
                        ex_h.at[pl.ds(HD0 * E + eb, CHB)], exa, sI),
                    pltpu.make_async_copy(
                        ex_h.at[pl.ds(HD1 * E + eb, CHB)], exb, sI))

        def idx_async(k, b):
            for d in idx_copies(k, b):
                d.start()

        def gather(k, b):
            for d in idx_copies(k, b):
                d.wait()
            srcB, _dstB, _ea, _eb, rows, sG, _sI = bufs[b]
            pltpu.async_copy(h_t.at[srcB], rows, sG)

        def process(b):
            srcB, dstB, exa, exb, rows, sG, _sI = bufs[b]
            pltpu.make_async_copy(h_t.at[srcB], rows, sG).wait()

            def scale16(g, _):
                off = _ds16(g)
                ex16a = exa[off]
                ex16b = exb[off]
                for j in range(16):
                    ea = ex16a[j]
                    eb_ = ex16b[j]
                    r = g * 16 + j
                    for jj in range(C // 16):
                        rows[r, _ds16(jj)] = rows[r, _ds16(jj)] * ea
                    for jj in range(C // 16):
                        sl = pl.ds(pl.multiple_of(C + jj * 16, 16), 16)
                        rows[r, sl] = rows[r, sl] * eb_
                return 0
            lax.fori_loop(0, CHB // 16, scale16, 0)
            pltpu.sync_copy(rows, acc_sp.at[dstB], add=True)

        idx_async(0, 0)
        gather(0, 0)
        idx_async(1, 1)

        def outer(ko, _):
            gather(2 * ko + 1, 1)
            process(0)

            @pl.when(ko < NC2 - 1)
            def _():
                idx_async(2 * ko + 2, 0)
            process(1)

            @pl.when(ko < NC2 - 1)
            def _():
                gather(2 * ko + 2, 0)
                idx_async(2 * ko + 3, 1)
            return 0
        lax.fori_loop(0, NC2, outer, 0)

        plsc.subcore_barrier()
        pltpu.sync_copy(acc_sp.at[pl.ds(nbase, SLN)],
                        agg_t.at[pl.ds(nbase, SLN)])

    @pl.when(c == 0)
    def _():
        run_core(hA, aggA, 0, 1)

    @pl.when(c == 1)
    def _():
        run_core(hB, aggB, 2, 3)


@functools.partial(
    pl.kernel,
    out_type=[pltpu.HBM((NP, 2 * C), jnp.float32),
              pltpu.HBM((NP, 2 * C), jnp.float32)],
    mesh=plsc.VectorSubcoreMesh(core_axis_name="c", subcore_axis_name="s"),
    compiler_params=pltpu.CompilerParams(needs_layout_passes=False),
    scratch_types=[
        pltpu.VMEM((CHB,), jnp.int32),
        pltpu.VMEM((CHB,), jnp.int32),
        pltpu.VMEM((CHB,), jnp.float32),
        pltpu.VMEM((CHB,), jnp.float32),
        pltpu.VMEM((CHB, 2 * C), jnp.float32),
        pltpu.VMEM((CHB,), jnp.int32),
        pltpu.VMEM((CHB,), jnp.int32),
        pltpu.VMEM((CHB,), jnp.float32),
        pltpu.VMEM((CHB,), jnp.float32),
        pltpu.VMEM((CHB, 2 * C), jnp.float32),
        pltpu.VMEM_SHARED((NP, 2 * C), jnp.float32),
        pltpu.SemaphoreType.DMA,
        pltpu.SemaphoreType.DMA,
        pltpu.SemaphoreType.DMA,
        pltpu.SemaphoreType.DMA,
    ],
)
def _sc_agg_kernel(*refs):
    _sc_agg_body(*refs)


def _gat_edge_sc(als, ald, exl, h_heads, src, dst):
    alsT = als.T.reshape(-1)
    aldT = ald.T.reshape(-1)
    exlT = exl.T.reshape(-1)
    ex, den = _sc_logit_kernel(src, dst, alsT, aldT, exlT)
    zrow = jnp.zeros((SLN, 2 * C), jnp.float32)
    aggA, aggB = _sc_agg_kernel(src, dst, ex,
                                h_heads[:, :2 * C], h_heads[:, 2 * C:], zrow)
    agg = jnp.concatenate([aggA, aggB], axis=1)
    return agg, den.reshape(H, NP).T


def kernel(x, edge_index, batch, W1, a_src1, a_dst1, b1, W2, a_src2, a_dst2,
           b2, W_res, b_res, W_fc, b_fc):
    f32 = jnp.float32
    x_p = jnp.zeros((NP, D), f32).at[:N].set(x)
    src = edge_index[0]
    dst = edge_index[1]

    eyeC = jnp.eye(H, dtype=f32)
    amat1 = jnp.concatenate(
        [jnp.einsum('hc,hk->hck', a_src1, eyeC).reshape(HC, H),
         jnp.einsum('hc,hk->hck', a_dst1, eyeC).reshape(HC, H)], axis=1)
    amat2 = jnp.concatenate(
        [jnp.einsum('hc,hk->hck', a_src2, eyeC).reshape(HC, H),
         jnp.einsum('hc,hk->hck', a_dst2, eyeC).reshape(HC, H)], axis=1)
    rmat = jnp.repeat(jnp.eye(H, dtype=f32), C, axis=1)

    Wcat = jnp.concatenate([W1, W_res], axis=1)
    bcat = jnp.concatenate([jnp.zeros((HC,), f32), b_res])[None, :]

    oh, aa1, exl1 = _stage_a(x_p, Wcat, bcat, amat1)
    h1 = oh[:, :HC]
    res = oh[:, HC:]

    agg1_f, den1_f = _gat_edge_sc(aa1[:, :H], aa1[:, H:], exl1, h1, src, dst)

    h2, aa2, exl2 = _stage_b(agg1_f, h1, exl1, den1_f, b1[None, :], rmat,
                             W2, amat2)

    agg2_f, den2_f = _gat_edge_sc(aa2[:, :H], aa2[:, H:], exl2, h2, src, dst)

    gids = jnp.arange(G, dtype=batch.dtype)
    onehot = (batch[None, :] == gids[:, None]).astype(f32)
    counts = onehot.sum(axis=1)
    ohw = onehot / jnp.maximum(counts, 1.0)[:, None]
    ohw = jnp.zeros((G, NP), f32).at[:, :N].set(ohw)

    out = _stage_c(agg2_f, h2, exl2, den2_f, res, ohw, b2[None, :], rmat,
                   W_fc, bfc_r := b_fc[None, :])
    return (out, jnp.array(1))


# speedup vs baseline: 89.2086x; 1.1482x over previous
import functools
import jax
import jax.numpy as jnp
from jax import lax
from jax.experimental import pallas as pl
from jax.experimental.pallas import tpu as pltpu
from jax.experimental.pallas import tpu_sc as plsc

N = 10000
E = 320000
D = 128
H = 4
C = 64
HC = 256
G = 64
OUT = 128

NP = 10240
BN = 1024
NB = NP // BN


def _stage_a_body(x_ref, wcat_ref, bcat_ref, amat_ref, oh_ref, aa_ref, exl_ref):
    xb = x_ref[...]
    hres = jnp.dot(xb, wcat_ref[...], preferred_element_type=jnp.float32)
    hres = hres + bcat_ref[...]
    oh_ref[...] = hres
    aa = jnp.dot(hres[:, :HC], amat_ref[...], preferred_element_type=jnp.float32)
    aa_ref[...] = aa
    s = aa[:, :H] + aa[:, H:]
    s = jnp.where(s >= 0, s, 0.2 * s)
    exl_ref[...] = jnp.exp(s)


def _stage_a(x_p, Wcat, bcat, amat):
    return pl.pallas_call(
        _stage_a_body,
        grid=(NB,),
        in_specs=[
            pl.BlockSpec((BN, D), lambda i: (i, 0)),
            pl.BlockSpec((D, 2 * HC), lambda i: (0, 0)),
            pl.BlockSpec((1, 2 * HC), lambda i: (0, 0)),
            pl.BlockSpec((HC, 2 * H), lambda i: (0, 0)),
        ],
        out_specs=[
            pl.BlockSpec((BN, 2 * HC), lambda i: (i, 0)),
            pl.BlockSpec((BN, 2 * H), lambda i: (i, 0)),
            pl.BlockSpec((BN, H), lambda i: (i, 0)),
        ],
        out_shape=[
            jax.ShapeDtypeStruct((NP, 2 * HC), jnp.float32),
            jax.ShapeDtypeStruct((NP, 2 * H), jnp.float32),
            jax.ShapeDtypeStruct((NP, H), jnp.float32),
        ],
    )(x_p, Wcat, bcat, amat)


def _stage_b_body(agg_ref, h1_ref, exl_ref, den_ref, b1_ref, rmat_ref,
                  w2_ref, amat_ref, oh_ref, aa_ref, exl2_ref):
    exl_rep = jnp.dot(exl_ref[...], rmat_ref[...], preferred_element_type=jnp.float32)
    den_rep = jnp.dot(den_ref[...], rmat_ref[...], preferred_element_type=jnp.float32)
    num = agg_ref[...] + exl_rep * h1_ref[...]
    o1 = jnp.maximum(num / den_rep + b1_ref[...], 0.0)
    h2 = jnp.dot(o1, w2_ref[...], preferred_element_type=jnp.float32)
    oh_ref[...] = h2
    aa = jnp.dot(h2, amat_ref[...], preferred_element_type=jnp.float32)
    aa_ref[...] = aa
    s = aa[:, :H] + aa[:, H:]
    s = jnp.where(s >= 0, s, 0.2 * s)
    exl2_ref[...] = jnp.exp(s)


def _stage_b(agg, h1, exl, den, b1r, rmat, W2, amat):
    return pl.pallas_call(
        _stage_b_body,
        grid=(NB,),
        in_specs=[
            pl.BlockSpec((BN, HC), lambda i: (i, 0)),
            pl.BlockSpec((BN, HC), lambda i: (i, 0)),
            pl.BlockSpec((BN, H), lambda i: (i, 0)),
            pl.BlockSpec((BN, H), lambda i: (i, 0)),
            pl.BlockSpec((1, HC), lambda i: (0, 0)),
            pl.BlockSpec((H, HC), lambda i: (0, 0)),
            pl.BlockSpec((HC, HC), lambda i: (0, 0)),
            pl.BlockSpec((HC, 2 * H), lambda i: (0, 0)),
        ],
        out_specs=[
            pl.BlockSpec((BN, HC), lambda i: (i, 0)),
            pl.BlockSpec((BN, 2 * H), lambda i: (i, 0)),
            pl.BlockSpec((BN, H), lambda i: (i, 0)),
        ],
        out_shape=[
            jax.ShapeDtypeStruct((NP, HC), jnp.float32),
            jax.ShapeDtypeStruct((NP, 2 * H), jnp.float32),
            jax.ShapeDtypeStruct((NP, H), jnp.float32),
        ],
    )(agg, h1, exl, den, b1r, rmat, W2, amat)


def _stage_c_body(agg_ref, h2_ref, exl_ref, den_ref, res_ref, ohw_ref, b2_ref,
                  rmat_ref, wfc_ref, bfc_ref, out_ref, acc_ref):
    i = pl.program_id(0)

    @pl.when(i == 0)
    def _():
        acc_ref[...] = jnp.zeros_like(acc_ref)

    exl_rep = jnp.dot(exl_ref[...], rmat_ref[...], preferred_element_type=jnp.float32)
    den_rep = jnp.dot(den_ref[...], rmat_ref[...], preferred_element_type=jnp.float32)
    num = agg_ref[...] + exl_rep * h2_ref[...]
    hfin = jnp.maximum(num / den_rep + b2_ref[...], 0.0) + res_ref[...]
    acc_ref[...] += jnp.dot(ohw_ref[...], hfin, preferred_element_type=jnp.float32)

    @pl.when(i == NB - 1)
    def _():
        logits = jnp.dot(acc_ref[...], wfc_ref[...],
                         preferred_element_type=jnp.float32) + bfc_ref[...]
        m = jnp.max(logits, axis=1, keepdims=True)
        lse = jnp.log(jnp.sum(jnp.exp(logits - m), axis=1, keepdims=True)) + m
        out_ref[...] = logits - lse


def _stage_c(agg, h2, exl, den, res, ohw, b2r, rmat, Wfc, bfcr):
    return pl.pallas_call(
        _stage_c_body,
        grid=(NB,),
        in_specs=[
            pl.BlockSpec((BN, HC), lambda i: (i, 0)),
            pl.BlockSpec((BN, HC), lambda i: (i, 0)),
            pl.BlockSpec((BN, H), lambda i: (i, 0)),
            pl.BlockSpec((BN, H), lambda i: (i, 0)),
            pl.BlockSpec((BN, HC), lambda i: (i, 0)),
            pl.BlockSpec((G, BN), lambda i: (0, i)),
            pl.BlockSpec((1, HC), lambda i: (0, 0)),
            pl.BlockSpec((H, HC), lambda i: (0, 0)),
            pl.BlockSpec((HC, OUT), lambda i: (0, 0)),
            pl.BlockSpec((1, OUT), lambda i: (0, 0)),
        ],
        out_specs=pl.BlockSpec((G, OUT), lambda i: (0, 0)),
        out_shape=jax.ShapeDtypeStruct((G, OUT), jnp.float32),
        scratch_shapes=[pltpu.VMEM((G, HC), jnp.float32)],
    )(agg, h2, exl, den, res, ohw, b2r, rmat, Wfc, bfcr)


NT = 16
EPT = E // NT
SLN = NP // NT
CHA = 2000
CHB = 80


def _ds16(i):
    return pl.ds(pl.multiple_of(i * 16, 16), 16)


def _sc_logit_body(src_h, dst_h, als_h, ald_h, exl_h,
                   ex_o, den_o,
                   t_as0, t_ad0, t_as1, t_ad1, den_l0, den_l1,
                   src_v, dst_v, ex_v0, ex_v1, tmp_a, acc_a,
                   den_parts):
    c = lax.axis_index("c")
    s = lax.axis_index("s")
    ebase = s * EPT
    nbase = s * SLN

    def run_core(HD0, HD1):
        pltpu.sync_copy(als_h.at[pl.ds(HD0 * NP, NP)], t_as0)
        pltpu.sync_copy(ald_h.at[pl.ds(HD0 * NP, NP)], t_ad0)
        pltpu.sync_copy(als_h.at[pl.ds(HD1 * NP, NP)], t_as1)
        pltpu.sync_copy(ald_h.at[pl.ds(HD1 * NP, NP)], t_ad1)

        def zv(i, _):
            z = jnp.zeros((16,), jnp.float32)
            den_l0[_ds16(i)] = z
            den_l1[_ds16(i)] = z
            return 0
        lax.fori_loop(0, NP // 16, zv, 0)

        def chunk_a(k, _):
            pltpu.sync_copy(src_h.at[pl.ds(ebase + k * CHA, CHA)], src_v)
            pltpu.sync_copy(dst_h.at[pl.ds(ebase + k * CHA, CHA)], dst_v)

            def vec16(i, _):
                sl = _ds16(i)
                s16 = src_v[sl]
                d16 = dst_v[sl]
                for lh, (tas, tad, denl) in enumerate(
                        ((t_as0, t_ad0, den_l0), (t_as1, t_ad1, den_l1))):
                    e = (plsc.load_gather(tas, [s16])
                         + plsc.load_gather(tad, [d16]))
                    e = jnp.where(e >= 0, e, e * 0.2)
                    ex = jnp.exp(e)
                    (ex_v0 if lh == 0 else ex_v1)[_ds16(i)] = ex
                    plsc.addupdate_scatter(denl, [d16], ex)
                return 0
            lax.fori_loop(0, CHA // 16, vec16, 0)
            pltpu.sync_copy(ex_v0, ex_o.at[pl.ds(HD0 * E + ebase + k * CHA, CHA)])
            pltpu.sync_copy(ex_v1, ex_o.at[pl.ds(HD1 * E + ebase + k * CHA, CHA)])
            return 0
        lax.fori_loop(0, EPT // CHA, chunk_a, 0)

        pltpu.sync_copy(den_l0, den_parts.at[s, 0])
        pltpu.sync_copy(den_l1, den_parts.at[s, 1])
        plsc.subcore_barrier()
        for lh, HD in ((0, HD0), (1, HD1)):
            pltpu.sync_copy(exl_h.at[pl.ds(HD * NP + nbase, SLN)], acc_a)

            def red_t(t, _):
                pltpu.sync_copy(den_parts.at[t, lh, pl.ds(nbase, SLN)], tmp_a)

                def addv(i, _):
                    sl = _ds16(i)
                    acc_a[sl] = acc_a[sl] + tmp_a[sl]
                    return 0
                lax.fori_loop(0, SLN // 16, addv, 0)
                return 0
            lax.fori_loop(0, NT, red_t, 0)
            pltpu.sync_copy(acc_a, den_o.at[pl.ds(HD * NP + nbase, SLN)])

    @pl.when(c == 0)
    def _():
        run_core(0, 1)

    @pl.when(c == 1)
    def _():
        run_core(2, 3)


@functools.partial(
    pl.kernel,
    out_type=[pltpu.HBM((H * E,), jnp.float32),
              pltpu.HBM((H * NP,), jnp.float32)],
    mesh=plsc.VectorSubcoreMesh(core_axis_name="c", subcore_axis_name="s"),
    compiler_params=pltpu.CompilerParams(needs_layout_passes=False),
    scratch_types=[
        pltpu.VMEM((NP,), jnp.float32),
        pltpu.VMEM((NP,), jnp.float32),
        pltpu.VMEM((NP,), jnp.float32),
        pltpu.VMEM((NP,), jnp.float32),
        pltpu.VMEM((NP,), jnp.float32),
        pltpu.VMEM((NP,), jnp.float32),
        pltpu.VMEM((CHA,), jnp.int32),
        pltpu.VMEM((CHA,), jnp.int32),
        pltpu.VMEM((CHA,), jnp.float32),
        pltpu.VMEM((CHA,), jnp.float32),
        pltpu.VMEM((SLN,), jnp.float32),
        pltpu.VMEM((SLN,), jnp.float32),
        pltpu.VMEM_SHARED((NT, 2, NP), jnp.float32),
    ],
)
def _sc_logit_kernel(*refs):
    _sc_logit_body(*refs)


def _sc_agg_body(src_h, dst_h, ex_h, hA, hB, zrow_h,
                 aggA, aggB,
                 srcB0, dstB0, exa0, exb0, rows0,
                 srcB1, dstB1, exa1, exb1, rows1,
                 dstS0, dstS1,
                 acc_sp, semG0, semG1, semI0, semI1, semS0, semS1):
    c = lax.axis_index("c")
    s = lax.axis_index("s")
    ebase = s * EPT
    nbase = s * SLN
    NC2 = (EPT // CHB) // 2

    def run_core(h_t, agg_t, HD0, HD1):
        pltpu.sync_copy(zrow_h, acc_sp.at[pl.ds(nbase, SLN)])
        plsc.subcore_barrier()

        bufs = ((srcB0, dstB0, exa0, exb0, rows0, semG0, semI0, dstS0, semS0),
                (srcB1, dstB1, exa1, exb1, rows1, semG1, semI1, dstS1, semS1))

        def idx_copies(k, b):
            eb = ebase + k * CHB
            srcB, dstB, exa, exb, _rows, _sG, sI, _dS, _sS = bufs[b]
            return (pltpu.make_async_copy(src_h.at[pl.ds(eb, CHB)], srcB, sI),
                    pltpu.make_async_copy(dst_h.at[pl.ds(eb, CHB)], dstB, sI),
                    pltpu.make_async_copy(
                        ex_h.at[pl.ds(HD0 * E + eb, CHB)], exa, sI),
                    pltpu.make_async_copy(
                        ex_h.at[pl.ds(HD1 * E + eb, CHB)], exb, sI))

        def idx_async(k, b):
            for d in idx_copies(k, b):
                d.start()

        def drain_scatter(b):
            _s, _d, _ea, _eb, rows, _sG, _sI, dS, sS = bufs[b]
            pltpu.make_async_copy(rows, acc_sp.at[dS], sS).wait()

        def gather(k, b):
            for d in idx_copies(k, b):
                d.wait()
            srcB, _dstB, _ea, _eb, rows, sG, _sI, _dS, _sS = bufs[b]
            pltpu.async_copy(h_t.at[srcB], rows, sG)

        def process(b):
            srcB, dstB, exa, exb, rows, sG, _sI, dS, sS = bufs[b]
            pltpu.make_async_copy(h_t.at[srcB], rows, sG).wait()
            for q in range(CHB // 16):
                dS[_ds16(q)] = dstB[_ds16(q)]

            def scale16(g, _):
                off = _ds16(g)
                ex16a = exa[off]
                ex16b = exb[off]
                for j in range(16):
                    ea = ex16a[j]
                    eb_ = ex16b[j]
                    r = g * 16 + j
                    for jj in range(C // 16):
                        rows[r, _ds16(jj)] = rows[r, _ds16(jj)] * ea
                    for jj in range(C // 16):
                        sl = pl.ds(pl.multiple_of(C + jj * 16, 16), 16)
                        rows[r, sl] = rows[r, sl] * eb_
                return 0
            lax.fori_loop(0, CHB // 16, scale16, 0)
            pltpu.async_copy(rows, acc_sp.at[dS], sS, add=True)

        idx_async(0, 0)
        gather(0, 0)
        idx_async(1, 1)

        def outer(ko, _):
            @pl.when(ko > 0)
            def _():
                drain_scatter(1)
            gather(2 * ko + 1, 1)
            process(0)

            @pl.when(ko < NC2 - 1)
            def _():
                idx_async(2 * ko + 2, 0)
            process(1)

            @pl.when(ko < NC2 - 1)
            def _():
                drain_scatter(0)
                gather(2 * ko + 2, 0)
                idx_async(2 * ko + 3, 1)
            return 0
        lax.fori_loop(0, NC2, outer, 0)
        drain_scatter(0)
        drain_scatter(1)

        plsc.subcore_barrier()
        pltpu.sync_copy(acc_sp.at[pl.ds(nbase, SLN)],
                        agg_t.at[pl.ds(nbase, SLN)])

    @pl.when(c == 0)
    def _():
        run_core(hA, aggA, 0, 1)

    @pl.when(c == 1)
    def _():
        run_core(hB, aggB, 2, 3)


@functools.partial(
    pl.kernel,
    out_type=[pltpu.HBM((NP, 2 * C), jnp.float32),
              pltpu.HBM((NP, 2 * C), jnp.float32)],
    mesh=plsc.VectorSubcoreMesh(core_axis_name="c", subcore_axis_name="s"),
    compiler_params=pltpu.CompilerParams(needs_layout_passes=False),
    scratch_types=[
        pltpu.VMEM((CHB,), jnp.int32),
        pltpu.VMEM((CHB,), jnp.int32),
        pltpu.VMEM((CHB,), jnp.float32),
        pltpu.VMEM((CHB,), jnp.float32),
        pltpu.VMEM((CHB, 2 * C), jnp.float32),
        pltpu.VMEM((CHB,), jnp.int32),
        pltpu.VMEM((CHB,), jnp.int32),
        pltpu.VMEM((CHB,), jnp.float32),
        pltpu.VMEM((CHB,), jnp.float32),
        pltpu.VMEM((CHB, 2 * C), jnp.float32),
        pltpu.VMEM((CHB,), jnp.int32),
        pltpu.VMEM((CHB,), jnp.int32),
        pltpu.VMEM_SHARED((NP, 2 * C), jnp.float32),
        pltpu.SemaphoreType.DMA,
        pltpu.SemaphoreType.DMA,
        pltpu.SemaphoreType.DMA,
        pltpu.SemaphoreType.DMA,
        pltpu.SemaphoreType.DMA,
        pltpu.SemaphoreType.DMA,
    ],
)
def _sc_agg_kernel(*refs):
    _sc_agg_body(*refs)


def _gat_edge_sc(als, ald, exl, h_heads, src, dst):
    alsT = als.T.reshape(-1)
    aldT = ald.T.reshape(-1)
    exlT = exl.T.reshape(-1)
    ex, den = _sc_logit_kernel(src, dst, alsT, aldT, exlT)
    zrow = jnp.zeros((SLN, 2 * C), jnp.float32)
    aggA, aggB = _sc_agg_kernel(src, dst, ex,
                                h_heads[:, :2 * C], h_heads[:, 2 * C:], zrow)
    agg = jnp.concatenate([aggA, aggB], axis=1)
    return agg, den.reshape(H, NP).T


def kernel(x, edge_index, batch, W1, a_src1, a_dst1, b1, W2, a_src2, a_dst2,
           b2, W_res, b_res, W_fc, b_fc):
    f32 = jnp.float32
    x_p = jnp.zeros((NP, D), f32).at[:N].set(x)
    src = edge_index[0]
    dst = edge_index[1]

    eyeC = jnp.eye(H, dtype=f32)
    amat1 = jnp.concatenate(
        [jnp.einsum('hc,hk->hck', a_src1, eyeC).reshape(HC, H),
         jnp.einsum('hc,hk->hck', a_dst1, eyeC).reshape(HC, H)], axis=1)
    amat2 = jnp.concatenate(
        [jnp.einsum('hc,hk->hck', a_src2, eyeC).reshape(HC, H),
         jnp.einsum('hc,hk->hck', a_dst2, eyeC).reshape(HC, H)], axis=1)
    rmat = jnp.repeat(jnp.eye(H, dtype=f32), C, axis=1)

    Wcat = jnp.concatenate([W1, W_res], axis=1)
    bcat = jnp.concatenate([jnp.zeros((HC,), f32), b_res])[None, :]

    oh, aa1, exl1 = _stage_a(x_p, Wcat, bcat, amat1)
    h1 = oh[:, :HC]
    res = oh[:, HC:]

    agg1_f, den1_f = _gat_edge_sc(aa1[:, :H], aa1[:, H:], exl1, h1, src, dst)

    h2, aa2, exl2 = _stage_b(agg1_f, h1, exl1, den1_f, b1[None, :], rmat,
                             W2, amat2)

    agg2_f, den2_f = _gat_edge_sc(aa2[:, :H], aa2[:, H:], exl2, h2, src, dst)

    gids = jnp.arange(G, dtype=batch.dtype)
    onehot = (batch[None, :] == gids[:, None]).astype(f32)
    counts = onehot.sum(axis=1)
    ohw = onehot / jnp.maximum(counts, 1.0)[:, None]
    ohw = jnp.zeros((G, NP), f32).at[:, :N].set(ohw)

    out = _stage_c(agg2_f, h2, exl2, den2_f, res, ohw, b2[None, :], rmat,
                   W_fc, bfc_r := b_fc[None, :])
    return (out, jnp.array(1))
